# baseline (speedup 1.0000x reference)
import functools

import jax
import jax.numpy as jnp
from jax import lax
from jax.experimental import pallas as pl
from jax.experimental.pallas import tpu as pltpu
from jax.experimental.pallas import tpu_sc as plsc

N_USERS = 8192
N_ITEMS = 2048
BATCH = 1024


BK = 1024
NKB = N_USERS // BK
RCHUNK = 128
TS = 256


def _sim_softmax_body(a_ref, t_ref, out_ref, acc_ref):
    k = pl.program_id(0)

    @pl.when(k == 0)
    def _():
        acc_ref[...] = jnp.zeros_like(acc_ref)

    ab = a_ref[...].astype(jnp.bfloat16)
    for j in range(N_ITEMS // TS):
        hi = (j + 1) * TS
        acc_ref[0:hi, j * TS:hi] += lax.dot_general(
            ab[:, 0:hi], ab[:, j * TS:hi], (((0,), (0,)), ((), ())),
            preferred_element_type=jnp.float32)

    @pl.when(k == NKB - 1)
    def _():
        for ti in range(N_ITEMS // TS):
            for tj in range(ti + 1, N_ITEMS // TS):
                tile = acc_ref[ti * TS:(ti + 1) * TS, tj * TS:(tj + 1) * TS]
                acc_ref[tj * TS:(tj + 1) * TS, ti * TS:(ti + 1) * TS] = tile.T

        temp = t_ref[0, 0]

        def body(i, carry):
            r0 = i * RCHUNK
            sim = acc_ref[pl.ds(r0, RCHUNK), :]
            mask = sim > 0.0
            scaled = sim / temp
            rmax = jnp.max(
                jnp.where(mask, scaled, jnp.float32(-1e30)), axis=1,
                keepdims=True)
            e = jnp.where(mask, jnp.exp(scaled - rmax), 0.0)
            rsum = jnp.sum(e, axis=1, keepdims=True)
            out_ref[pl.ds(r0, RCHUNK), :] = jnp.where(
                rsum > 0.0, e / jnp.maximum(rsum, jnp.float32(1e-12)),
                0.0).astype(jnp.bfloat16)
            return carry

        lax.fori_loop(0, N_ITEMS // RCHUNK, body, 0)


def _sim_softmax(adj_mat, temp2d):
    return pl.pallas_call(
        _sim_softmax_body,
        grid=(NKB,),
        in_specs=[
            pl.BlockSpec((BK, N_ITEMS), lambda k: (k, 0)),
            pl.BlockSpec(memory_space=pltpu.SMEM),
        ],
        out_specs=pl.BlockSpec((N_ITEMS, N_ITEMS), lambda k: (0, 0)),
        out_shape=jax.ShapeDtypeStruct((N_ITEMS, N_ITEMS), jnp.bfloat16),
        scratch_shapes=[pltpu.VMEM((N_ITEMS, N_ITEMS), jnp.float32)],
    )(adj_mat, temp2d)


BN = 512


def _scores_body(p_ref, s_ref, o_ref):
    o_ref[...] = lax.dot_general(
        p_ref[...].astype(jnp.bfloat16), s_ref[...],
        (((1,), (0,)), ((), ())), preferred_element_type=jnp.float32)


def _matmul_scores(profiles, item_sim):
    return pl.pallas_call(
        _scores_body,
        grid=(N_ITEMS // BN,),
        in_specs=[
            pl.BlockSpec((BATCH, N_ITEMS), lambda n: (0, 0)),
            pl.BlockSpec((N_ITEMS, BN), lambda n: (0, n)),
        ],
        out_specs=pl.BlockSpec((BATCH, BN), lambda n: (0, n)),
        out_shape=jax.ShapeDtypeStruct((BATCH, N_ITEMS), jnp.float32),
    )(profiles, item_sim)


@functools.cache
def _make_gather():
    info = plsc.get_sparse_core_info()
    nw = info.num_cores * info.num_subcores
    bpw = BATCH // nw
    mesh = plsc.VectorSubcoreMesh(core_axis_name="c", subcore_axis_name="s")

    @functools.partial(
        pl.kernel, mesh=mesh,
        out_type=jax.ShapeDtypeStruct((BATCH, N_ITEMS), jnp.float32),
        scratch_types=[
            pltpu.VMEM((bpw,), jnp.int32),
            pltpu.VMEM((bpw, N_ITEMS), jnp.float32),
            pltpu.SemaphoreType.DMA,
        ],
    )
    def gather(table_hbm, idx_hbm, out_hbm, idx_v, rows_v, sem):
        wid = lax.axis_index("s") * info.num_cores + lax.axis_index("c")
        base = wid * bpw
        pltpu.sync_copy(idx_hbm.at[pl.ds(base, bpw)], idx_v)
        pltpu.async_copy(table_hbm.at[idx_v], rows_v, sem).wait()
        pltpu.sync_copy(rows_v, out_hbm.at[pl.ds(base, bpw)])

    return gather


def kernel(users, adj_mat, temperature):
    users = users.astype(jnp.int32)
    temp2d = jnp.reshape(temperature.astype(jnp.float32), (1, 1))
    profiles = jnp.take(adj_mat, users, axis=0)
    item_sim = _sim_softmax(adj_mat, temp2d)
    return _matmul_scores(profiles, item_sim)


# speedup vs baseline: 1.2554x; 1.2554x over previous
import functools

import jax
import jax.numpy as jnp
from jax import lax
from jax.experimental import pallas as pl
from jax.experimental.pallas import tpu as pltpu
from jax.experimental.pallas import tpu_sc as plsc

N_USERS = 8192
N_ITEMS = 2048
BATCH = 1024


BK = 1024
NKB = N_USERS // BK
RCHUNK = 128
TS = 256


def _sim_softmax_body(a_ref, t_ref, out_ref, acc_ref):
    k = pl.program_id(0)

    @pl.when(k == 0)
    def _():
        acc_ref[...] = jnp.zeros_like(acc_ref)

    ab = a_ref[...].astype(jnp.bfloat16)
    for j in range(N_ITEMS // TS):
        hi = (j + 1) * TS
        acc_ref[0:hi, j * TS:hi] += lax.dot_general(
            ab[:, 0:hi], ab[:, j * TS:hi], (((0,), (0,)), ((), ())),
            preferred_element_type=jnp.float32)

    @pl.when(k == NKB - 1)
    def _():
        temp = t_ref[0, 0]
        nt = N_ITEMS // TS
        for i in range(nt):
            parts = []
            for t in range(nt):
                if t < i:
                    parts.append(
                        acc_ref[t * TS:(t + 1) * TS, i * TS:(i + 1) * TS].T)
                else:
                    parts.append(
                        acc_ref[i * TS:(i + 1) * TS, t * TS:(t + 1) * TS])
            sim = jnp.concatenate(parts, axis=1)
            rmax = jnp.max(sim, axis=1, keepdims=True)
            e = jnp.where(sim > 0.0, jnp.exp((sim - rmax) / temp), 0.0)
            rsum = jnp.sum(e, axis=1, keepdims=True)
            out_ref[i * TS:(i + 1) * TS, :] = jnp.where(
                rsum > 0.0, e / jnp.maximum(rsum, jnp.float32(1e-12)),
                0.0).astype(jnp.bfloat16)


def _sim_softmax(adj_mat, temp2d):
    return pl.pallas_call(
        _sim_softmax_body,
        grid=(NKB,),
        in_specs=[
            pl.BlockSpec((BK, N_ITEMS), lambda k: (k, 0)),
            pl.BlockSpec(memory_space=pltpu.SMEM),
        ],
        out_specs=pl.BlockSpec((N_ITEMS, N_ITEMS), lambda k: (0, 0)),
        out_shape=jax.ShapeDtypeStruct((N_ITEMS, N_ITEMS), jnp.bfloat16),
        scratch_shapes=[pltpu.VMEM((N_ITEMS, N_ITEMS), jnp.float32)],
    )(adj_mat, temp2d)


BN = 512


def _scores_body(p_ref, s_ref, o_ref):
    o_ref[...] = lax.dot_general(
        p_ref[...].astype(jnp.bfloat16), s_ref[...],
        (((1,), (0,)), ((), ())), preferred_element_type=jnp.float32)


def _matmul_scores(profiles, item_sim):
    return pl.pallas_call(
        _scores_body,
        grid=(N_ITEMS // BN,),
        in_specs=[
            pl.BlockSpec((BATCH, N_ITEMS), lambda n: (0, 0)),
            pl.BlockSpec((N_ITEMS, BN), lambda n: (0, n)),
        ],
        out_specs=pl.BlockSpec((BATCH, BN), lambda n: (0, n)),
        out_shape=jax.ShapeDtypeStruct((BATCH, N_ITEMS), jnp.float32),
    )(profiles, item_sim)


@functools.cache
def _make_gather():
    info = plsc.get_sparse_core_info()
    nw = info.num_cores * info.num_subcores
    bpw = BATCH // nw
    mesh = plsc.VectorSubcoreMesh(core_axis_name="c", subcore_axis_name="s")

    @functools.partial(
        pl.kernel, mesh=mesh,
        out_type=jax.ShapeDtypeStruct((BATCH, N_ITEMS), jnp.float32),
        scratch_types=[
            pltpu.VMEM((bpw,), jnp.int32),
            pltpu.VMEM((bpw, N_ITEMS), jnp.float32),
            pltpu.SemaphoreType.DMA,
        ],
    )
    def gather(table_hbm, idx_hbm, out_hbm, idx_v, rows_v, sem):
        wid = lax.axis_index("s") * info.num_cores + lax.axis_index("c")
        base = wid * bpw
        pltpu.sync_copy(idx_hbm.at[pl.ds(base, bpw)], idx_v)
        pltpu.async_copy(table_hbm.at[idx_v], rows_v, sem).wait()
        pltpu.sync_copy(rows_v, out_hbm.at[pl.ds(base, bpw)])

    return gather


def kernel(users, adj_mat, temperature):
    users = users.astype(jnp.int32)
    temp2d = jnp.reshape(temperature.astype(jnp.float32), (1, 1))
    profiles = _make_gather()(adj_mat, users)
    item_sim = _sim_softmax(adj_mat, temp2d)
    return _matmul_scores(profiles, item_sim)
